# baseline (speedup 1.0000x reference)
import functools

import jax
import jax.numpy as jnp
from jax.experimental import pallas as pl

B, S, D_MODEL = 4, 4096, 4096
E, K = 64, 8
CAPACITY_FACTOR = 1.25
LB_WEIGHT = 0.01

_TILE = 512


def _logits_kernel(x_ref, wt_ref, out_ref):
    out_ref[...] = jnp.dot(x_ref[...], wt_ref[...],
                           preferred_element_type=jnp.float32)


def _router_logits(x_flat, wt):
    n, d = x_flat.shape
    e = wt.shape[1]
    grid = (n // _TILE,)
    return pl.pallas_call(
        _logits_kernel,
        grid=grid,
        in_specs=[
            pl.BlockSpec((_TILE, d), lambda i: (i, 0)),
            pl.BlockSpec((d, e), lambda i: (0, 0)),
        ],
        out_specs=pl.BlockSpec((_TILE, e), lambda i: (i, 0)),
        out_shape=jax.ShapeDtypeStruct((n, e), jnp.float32),
    )(x_flat, wt)


def kernel(x, W):
    b, s, d = x.shape
    N = b * s
    n_exp = W.shape[0]
    k = K
    x_flat = x.reshape(N, d)
    logits = _router_logits(x_flat, W.T)
    gates = jax.nn.softmax(logits, axis=-1)
    actual_k = min(k, n_exp)
    topk_val_tok, topk_idx_tok = jax.lax.top_k(gates, actual_k)
    rows = jnp.arange(N)[:, None]
    mask = jnp.zeros((N, n_exp), gates.dtype).at[rows, topk_idx_tok].set(1.0)
    capacity = max(1, int(CAPACITY_FACTOR * (N * k) / n_exp))
    capacity = min(capacity, N)
    scores = jnp.where(mask <= 0, -jnp.inf, gates)
    top_val_ce_t, top_idx_ce_t = jax.lax.top_k(scores.T, capacity)
    top_val_ce = top_val_ce_t.T
    top_idx_ce = top_idx_ce_t.T
    valid_ce = jnp.isfinite(top_val_ce)
    e_grid = jnp.broadcast_to(jnp.arange(n_exp)[None, :], top_idx_ce.shape)
    kept_f = jnp.zeros((N, n_exp), jnp.float32).at[top_idx_ce, e_grid].max(
        valid_ce.astype(jnp.float32))
    kept_mask = kept_f > 0
    combine = gates * kept_mask.astype(gates.dtype)
    denom = jnp.clip(combine.sum(axis=-1, keepdims=True), 1e-09, None)
    combine = combine / denom
    p = gates.mean(axis=0)
    assignment_fraction = kept_mask.astype(gates.dtype).mean(axis=0) / float(max(k, 1))
    aux_lb = n_exp * (p * assignment_fraction).sum() * LB_WEIGHT
    z = jax.scipy.special.logsumexp(logits, axis=-1)
    aux_z = jnp.mean(jnp.square(z)) * 0.0
    ent = -(jnp.log(jnp.clip(gates, 1e-09, None)) * gates).sum(axis=-1).mean()
    aux_entropy = -0.0 * ent
    aux_margin = jnp.asarray(0.0, dtype=logits.dtype)
    return (combine, kept_mask, top_idx_ce, valid_ce, aux_lb, aux_z,
            aux_entropy, aux_margin, gates, logits)


# speedup vs baseline: 2.3210x; 2.3210x over previous
import jax
import jax.numpy as jnp
from jax.experimental import pallas as pl

B, S, D_MODEL = 4, 4096, 4096
E, K = 64, 8
CAPACITY_FACTOR = 1.25
LB_WEIGHT = 0.01

_TILE = 512


_KCHUNK = 1024


def _logits_kernel(x_ref, w_ref, logits_ref):
    d = x_ref.shape[1]
    dn = (((1,), (1,)), ((), ()))
    acc = jax.lax.dot_general(x_ref[:, 0:_KCHUNK], w_ref[:, 0:_KCHUNK],
                              dimension_numbers=dn,
                              preferred_element_type=jnp.float32)
    for kk in range(_KCHUNK, d, _KCHUNK):
        acc = acc + jax.lax.dot_general(
            x_ref[:, kk:kk + _KCHUNK], w_ref[:, kk:kk + _KCHUNK],
            dimension_numbers=dn, preferred_element_type=jnp.float32)
    logits_ref[...] = acc


def _router(x_flat, w):
    n, d = x_flat.shape
    e = w.shape[0]
    grid = (n // _TILE,)
    return pl.pallas_call(
        _logits_kernel,
        grid=grid,
        in_specs=[
            pl.BlockSpec((_TILE, d), lambda i: (i, 0)),
            pl.BlockSpec((e, d), lambda i: (0, 0)),
        ],
        out_specs=pl.BlockSpec((_TILE, e), lambda i: (i, 0)),
        out_shape=jax.ShapeDtypeStruct((n, e), jnp.float32),
    )(x_flat, w)


def _topk_mask_kernel(gates_ref, scores_ref):
    g = gates_ref[...]
    lanes = jax.lax.broadcasted_iota(jnp.int32, g.shape, 1)
    work = g
    mask = jnp.zeros(g.shape, jnp.bool_)
    for _ in range(K):
        m = jnp.max(work, axis=-1, keepdims=True)
        eq = work == m
        first = jnp.min(jnp.where(eq, lanes, g.shape[1]), axis=-1,
                        keepdims=True)
        hit = lanes == first
        mask = jnp.logical_or(mask, hit)
        work = jnp.where(hit, -1.0, work)
    scores_ref[...] = jnp.where(mask, g, -jnp.inf)


def _scores(gates):
    n, e = gates.shape
    grid = (n // _TILE,)
    return pl.pallas_call(
        _topk_mask_kernel,
        grid=grid,
        in_specs=[pl.BlockSpec((_TILE, e), lambda i: (i, 0))],
        out_specs=pl.BlockSpec((_TILE, e), lambda i: (i, 0)),
        out_shape=jax.ShapeDtypeStruct((n, e), jnp.float32),
    )(gates)


def _combine_kernel(gates_ref, scores_ref, thresh_ref, combine_ref, kept_ref,
                    psum_ref, ksum_ref):
    i = pl.program_id(0)
    g = gates_ref[...]
    kept = scores_ref[...] >= thresh_ref[...]
    kf = kept.astype(jnp.float32)
    kept_ref[...] = kf
    cw = g * kf
    denom = jnp.clip(jnp.sum(cw, axis=-1, keepdims=True), 1e-09, None)
    combine_ref[...] = cw / denom

    @pl.when(i == 0)
    def _init():
        psum_ref[...] = jnp.zeros_like(psum_ref)
        ksum_ref[...] = jnp.zeros_like(ksum_ref)

    psum_ref[...] += jnp.sum(g, axis=0, keepdims=True)
    ksum_ref[...] += jnp.sum(kf, axis=0, keepdims=True)


def _combine(gates, scores, thresh):
    n, e = gates.shape
    grid = (n // _TILE,)
    return pl.pallas_call(
        _combine_kernel,
        grid=grid,
        in_specs=[
            pl.BlockSpec((_TILE, e), lambda i: (i, 0)),
            pl.BlockSpec((_TILE, e), lambda i: (i, 0)),
            pl.BlockSpec((1, e), lambda i: (0, 0)),
        ],
        out_specs=[
            pl.BlockSpec((_TILE, e), lambda i: (i, 0)),
            pl.BlockSpec((_TILE, e), lambda i: (i, 0)),
            pl.BlockSpec((1, e), lambda i: (0, 0)),
            pl.BlockSpec((1, e), lambda i: (0, 0)),
        ],
        out_shape=[
            jax.ShapeDtypeStruct((n, e), jnp.float32),
            jax.ShapeDtypeStruct((n, e), jnp.float32),
            jax.ShapeDtypeStruct((1, e), jnp.float32),
            jax.ShapeDtypeStruct((1, e), jnp.float32),
        ],
    )(gates, scores, thresh)


def kernel(x, W):
    b, s, d = x.shape
    N = b * s
    n_exp = W.shape[0]
    x_flat = x.reshape(N, d)
    logits = _router(x_flat, W)
    gates = jax.nn.softmax(logits, axis=-1)

    scores = _scores(gates)

    capacity = max(1, int(CAPACITY_FACTOR * (N * K) / n_exp))
    capacity = min(capacity, N)
    top_val_ce_t, top_idx_ce_t = jax.lax.top_k(scores.T, capacity)
    top_val_ce = top_val_ce_t.T
    top_idx_ce = top_idx_ce_t.T
    valid_ce = jnp.isfinite(top_val_ce)
    thresh = jnp.min(jnp.where(valid_ce, top_val_ce, jnp.inf), axis=0,
                     keepdims=True)

    combine, kept_f, psum, ksum = _combine(gates, scores, thresh)
    kept_mask = kept_f > 0

    p_times_af = (psum[0] / N) * (ksum[0] / (N * float(K)))
    aux_lb = n_exp * jnp.sum(p_times_af) * LB_WEIGHT
    aux_z = jnp.asarray(0.0, jnp.float32)
    aux_entropy = jnp.asarray(-0.0, jnp.float32)
    aux_margin = jnp.asarray(0.0, jnp.float32)
    return (combine, kept_mask, top_idx_ce, valid_ce, aux_lb, aux_z,
            aux_entropy, aux_margin, gates, logits)
